# Initial kernel scaffold; baseline (speedup 1.0000x reference)
import jax
import jax.numpy as jnp
from jax.experimental import pallas as pl


def kernel(xyz, points, layer_id, W0, b0, gamma0, beta0, W1, b1, gamma1, beta1, W2, b2, gamma2, beta2):
    raise NotImplementedError("write your pallas kernel here")


# speedup vs baseline: 7.3873x; 7.3873x over previous
import dataclasses
import functools

import jax
import jax.numpy as jnp
from jax import lax
from jax.experimental import pallas as pl
from jax.experimental.pallas import tpu as pltpu
from jax.experimental.pallas import tpu_sc as plsc

N = 16384
S = 2048
K = 64
R2 = 0.2 * 0.2
COUT = 256
L = 16
NW = 32
CPW = S // NW
BUF = 96


def _sc_compiler_params():
    cp = pltpu.CompilerParams()
    if "needs_layout_passes" in pltpu.CompilerParams.__dataclass_fields__:
        cp = dataclasses.replace(cp, needs_layout_passes=False)
    return cp


_RB = 2048


def _layer_body(x, w, b, g, be, out):
    c = w.shape[1]
    s = jnp.zeros((1, c), jnp.float32)
    q = jnp.zeros((1, c), jnp.float32)
    for i in range(N // _RB):
        y = jnp.dot(x[pl.ds(i * _RB, _RB), :], w[...],
                    preferred_element_type=jnp.float32,
                    precision=lax.Precision.HIGHEST) + b[...]
        out[pl.ds(i * _RB, _RB), :] = y
        s = s + jnp.sum(y, axis=0, keepdims=True)
        q = q + jnp.sum(y * y, axis=0, keepdims=True)
    mu = s * (1.0 / N)
    va = q * (1.0 / N) - mu * mu
    scale = lax.rsqrt(va + 1e-5) * g[...]
    shift = be[...] - mu * scale
    for i in range(N // _RB):
        blk = out[pl.ds(i * _RB, _RB), :]
        out[pl.ds(i * _RB, _RB), :] = jnp.maximum(blk * scale + shift, 0.0)


def _mlp(ftT, params):
    h = ftT
    for (W, b, g, be) in params:
        h = pl.pallas_call(
            _layer_body,
            out_shape=jax.ShapeDtypeStruct((N, W.shape[0]), jnp.float32),
        )(h, W.T, b.reshape(1, -1), g.reshape(1, -1), be.reshape(1, -1))
    return h


def _ballquery(px, py, pz, cx, cy, cz):
    mesh = plsc.VectorSubcoreMesh(core_axis_name="c", subcore_axis_name="s")

    @functools.partial(
        pl.kernel,
        out_type=jax.ShapeDtypeStruct((S * K,), jnp.int32),
        mesh=mesh,
        scratch_types=[
            pltpu.VMEM((N,), jnp.float32),
            pltpu.VMEM((N,), jnp.float32),
            pltpu.VMEM((N,), jnp.float32),
            pltpu.VMEM((N,), jnp.float32),
            pltpu.VMEM((CPW,), jnp.float32),
            pltpu.VMEM((CPW,), jnp.float32),
            pltpu.VMEM((CPW,), jnp.float32),
            pltpu.VMEM((BUF,), jnp.int32),
            pltpu.VMEM((CPW * K,), jnp.int32),
        ],
        compiler_params=_sc_compiler_params(),
    )
    def bq(px_h, py_h, pz_h, cx_h, cy_h, cz_h, out_h,
           xv, yv, zv, x2v, cxv, cyv, czv, buf, outv):
        wid = lax.axis_index("s") * 2 + lax.axis_index("c")
        base = wid * CPW
        pltpu.sync_copy(px_h, xv)
        pltpu.sync_copy(py_h, yv)
        pltpu.sync_copy(pz_h, zv)
        pltpu.sync_copy(cx_h.at[pl.ds(base, CPW)], cxv)
        pltpu.sync_copy(cy_h.at[pl.ds(base, CPW)], cyv)
        pltpu.sync_copy(cz_h.at[pl.ds(base, CPW)], czv)
        lane = lax.iota(jnp.int32, 16)

        def _bf(v):
            u = plsc.bitcast(v, jnp.int32)
            r = (u + 0x7FFF + ((u >> 16) & 1)) & ~0xFFFF
            return plsc.bitcast(r, jnp.float32)

        @pl.loop(0, N // L)
        def _pre(v):
            s0 = v * L
            xf = xv[pl.ds(s0, L)]
            yf = yv[pl.ds(s0, L)]
            zf = zv[pl.ds(s0, L)]
            x2v[pl.ds(s0, L)] = (xf * xf + yf * yf) + zf * zf
            xv[pl.ds(s0, L)] = _bf(xf)
            yv[pl.ds(s0, L)] = _bf(yf)
            zv[pl.ds(s0, L)] = _bf(zf)

        @pl.loop(0, CPW)
        def _center(ci):
            ci16 = jnp.full((L,), ci, jnp.int32)
            cxs = plsc.load_gather(cxv, [ci16])
            cys = plsc.load_gather(cyv, [ci16])
            czs = plsc.load_gather(czv, [ci16])
            c2s = (cxs * cxs + cys * cys) + czs * czs
            cbx, cby, cbz = _bf(cxs), _bf(cys), _bf(czs)

            def scan(v, cursor):
                s0 = v * L
                m = (xv[pl.ds(s0, L)] * cbx + yv[pl.ds(s0, L)] * cby
                     + zv[pl.ds(s0, L)] * cbz)
                d2 = (c2s + x2v[pl.ds(s0, L)]) - 2.0 * m
                hit = d2 <= R2

                @pl.when(cursor < K)
                def _():
                    plsc.store_compressed(buf.at[pl.ds(cursor, L)],
                                          lane + s0, mask=hit)

                return cursor + jnp.sum(hit.astype(jnp.int32))

            cursor = lax.fori_loop(0, N // L, scan, jnp.int32(0))
            cnt = jnp.minimum(cursor, K)
            for k in range(K // L):
                j = lane + (k * L)
                sel = jnp.where(j < cnt, j, 0)
                outv[pl.ds(ci * K + k * L, L)] = plsc.load_gather(buf, [sel])

        pltpu.sync_copy(outv, out_h.at[pl.ds(base * K, CPW * K)])

    return bq(px, py, pz, cx, cy, cz)


def _gathermax(featT, idxf):
    mesh = plsc.VectorSubcoreMesh(core_axis_name="c", subcore_axis_name="s")

    @functools.partial(
        pl.kernel,
        out_type=jax.ShapeDtypeStruct((S * COUT,), jnp.float32),
        mesh=mesh,
        scratch_types=[
            pltpu.VMEM((CPW * K,), jnp.int32),
            pltpu.VMEM((K, COUT), jnp.float32),
            pltpu.VMEM((CPW * COUT,), jnp.float32),
            pltpu.SemaphoreType.DMA,
        ],
        compiler_params=_sc_compiler_params(),
    )
    def gm(feat_h, idx_h, out_h, idxv, rows, outv, sem):
        wid = lax.axis_index("s") * 2 + lax.axis_index("c")
        base = wid * CPW
        pltpu.sync_copy(idx_h.at[pl.ds(base * K, CPW * K)], idxv)

        @pl.loop(0, CPW)
        def _center(ci):
            pltpu.async_copy(feat_h.at[idxv.at[pl.ds(ci * K, K)]],
                             rows, sem).wait()
            for k in range(COUT // L):
                def mbody(j, acc):
                    return jnp.maximum(acc, rows[j, pl.ds(k * L, L)])

                acc = lax.fori_loop(1, K, mbody, rows[0, pl.ds(k * L, L)])
                outv[pl.ds(ci * COUT + k * L, L)] = acc

        pltpu.sync_copy(outv, out_h.at[pl.ds(base * COUT, CPW * COUT)])

    return gm(featT, idxf)


def kernel(xyz, points, layer_id, W0, b0, gamma0, beta0, W1, b1, gamma1,
           beta1, W2, b2, gamma2, beta2):
    del layer_id
    px, py, pz = xyz[0, 0], xyz[0, 1], xyz[0, 2]
    stride = N // S
    cx, cy, cz = px[::stride], py[::stride], pz[::stride]
    new_xyz = xyz[:, :, ::stride]

    ftT = jnp.concatenate([xyz[0], points[0]], axis=0).T
    featT = _mlp(ftT, [(W0, b0, gamma0, beta0), (W1, b1, gamma1, beta1),
                       (W2, b2, gamma2, beta2)])

    idxf = _ballquery(px, py, pz, cx, cy, cz)
    gout = _gathermax(featT, idxf)
    new_points = gout.reshape(S, COUT).T[None]
    return (new_xyz, new_points)


# speedup vs baseline: 7.8576x; 1.0637x over previous
import dataclasses
import functools

import jax
import jax.numpy as jnp
from jax import lax
from jax.experimental import pallas as pl
from jax.experimental.pallas import tpu as pltpu
from jax.experimental.pallas import tpu_sc as plsc

N = 16384
S = 2048
K = 64
R2 = 0.2 * 0.2
COUT = 256
L = 16
NW = 32
CPW = S // NW
BUF = 96


def _sc_compiler_params():
    cp = pltpu.CompilerParams()
    if "needs_layout_passes" in pltpu.CompilerParams.__dataclass_fields__:
        cp = dataclasses.replace(cp, needs_layout_passes=False)
    return cp


_RB = 2048


def _layer_body(x, w, b, g, be, out):
    c = w.shape[1]
    s = jnp.zeros((1, c), jnp.float32)
    q = jnp.zeros((1, c), jnp.float32)
    for i in range(N // _RB):
        y = jnp.dot(x[pl.ds(i * _RB, _RB), :], w[...],
                    preferred_element_type=jnp.float32,
                    precision=lax.Precision.HIGHEST) + b[...]
        out[pl.ds(i * _RB, _RB), :] = y
        s = s + jnp.sum(y, axis=0, keepdims=True)
        q = q + jnp.sum(y * y, axis=0, keepdims=True)
    mu = s * (1.0 / N)
    va = q * (1.0 / N) - mu * mu
    scale = lax.rsqrt(va + 1e-5) * g[...]
    shift = be[...] - mu * scale
    for i in range(N // _RB):
        blk = out[pl.ds(i * _RB, _RB), :]
        out[pl.ds(i * _RB, _RB), :] = jnp.maximum(blk * scale + shift, 0.0)


def _mlp(ftT, params):
    h = ftT
    for (W, b, g, be) in params:
        h = pl.pallas_call(
            _layer_body,
            out_shape=jax.ShapeDtypeStruct((N, W.shape[0]), jnp.float32),
        )(h, W.T, b.reshape(1, -1), g.reshape(1, -1), be.reshape(1, -1))
    return h


def _ballquery(px, py, pz, cx, cy, cz):
    mesh = plsc.VectorSubcoreMesh(core_axis_name="c", subcore_axis_name="s")

    @functools.partial(
        pl.kernel,
        out_type=jax.ShapeDtypeStruct((S * K,), jnp.int32),
        mesh=mesh,
        scratch_types=[
            pltpu.VMEM((N,), jnp.float32),
            pltpu.VMEM((N,), jnp.float32),
            pltpu.VMEM((N,), jnp.float32),
            pltpu.VMEM((N,), jnp.float32),
            pltpu.VMEM((CPW,), jnp.float32),
            pltpu.VMEM((CPW,), jnp.float32),
            pltpu.VMEM((CPW,), jnp.float32),
            pltpu.VMEM((BUF,), jnp.int32),
            pltpu.VMEM((CPW * K,), jnp.int32),
        ],
        compiler_params=_sc_compiler_params(),
    )
    def bq(px_h, py_h, pz_h, cx_h, cy_h, cz_h, out_h,
           xv, yv, zv, x2v, cxv, cyv, czv, buf, outv):
        wid = lax.axis_index("s") * 2 + lax.axis_index("c")
        base = wid * CPW
        pltpu.sync_copy(px_h, xv)
        pltpu.sync_copy(py_h, yv)
        pltpu.sync_copy(pz_h, zv)
        pltpu.sync_copy(cx_h.at[pl.ds(base, CPW)], cxv)
        pltpu.sync_copy(cy_h.at[pl.ds(base, CPW)], cyv)
        pltpu.sync_copy(cz_h.at[pl.ds(base, CPW)], czv)
        lane = lax.iota(jnp.int32, 16)

        def _bf(v):
            u = plsc.bitcast(v, jnp.int32)
            r = (u + 0x7FFF + ((u >> 16) & 1)) & ~0xFFFF
            return plsc.bitcast(r, jnp.float32)

        @pl.loop(0, N // L)
        def _pre(v):
            s0 = v * L
            xf = xv[pl.ds(s0, L)]
            yf = yv[pl.ds(s0, L)]
            zf = zv[pl.ds(s0, L)]
            x2v[pl.ds(s0, L)] = (xf * xf + yf * yf) + zf * zf
            xv[pl.ds(s0, L)] = _bf(xf)
            yv[pl.ds(s0, L)] = _bf(yf)
            zv[pl.ds(s0, L)] = _bf(zf)

        @pl.loop(0, CPW)
        def _center(ci):
            ci16 = jnp.full((L,), ci, jnp.int32)
            cxs = plsc.load_gather(cxv, [ci16])
            cys = plsc.load_gather(cyv, [ci16])
            czs = plsc.load_gather(czv, [ci16])
            c2s = (cxs * cxs + cys * cys) + czs * czs
            cbx, cby, cbz = _bf(cxs), _bf(cys), _bf(czs)

            def scan(v, cursor):
                s0 = v * L
                m = (xv[pl.ds(s0, L)] * cbx + yv[pl.ds(s0, L)] * cby
                     + zv[pl.ds(s0, L)] * cbz)
                d2 = (c2s + x2v[pl.ds(s0, L)]) - 2.0 * m
                hit = d2 <= R2
                off = jnp.minimum(cursor, K)
                plsc.store_compressed(buf.at[pl.ds(off, L)],
                                      lane + s0, mask=hit)
                return cursor + plsc.all_reduce_population_count(hit)[0]

            cursor = lax.fori_loop(0, N // L, scan, jnp.int32(0),
                                   unroll=2)
            cnt = jnp.minimum(cursor, K)
            for k in range(K // L):
                j = lane + (k * L)
                sel = jnp.where(j < cnt, j, 0)
                outv[pl.ds(ci * K + k * L, L)] = plsc.load_gather(buf, [sel])

        pltpu.sync_copy(outv, out_h.at[pl.ds(base * K, CPW * K)])

    return bq(px, py, pz, cx, cy, cz)


def _gathermax(featT, idxf):
    mesh = plsc.VectorSubcoreMesh(core_axis_name="c", subcore_axis_name="s")

    @functools.partial(
        pl.kernel,
        out_type=jax.ShapeDtypeStruct((S * COUT,), jnp.float32),
        mesh=mesh,
        scratch_types=[
            pltpu.VMEM((CPW * K,), jnp.int32),
            pltpu.VMEM((K, COUT), jnp.float32),
            pltpu.VMEM((CPW * COUT,), jnp.float32),
            pltpu.SemaphoreType.DMA,
        ],
        compiler_params=_sc_compiler_params(),
    )
    def gm(feat_h, idx_h, out_h, idxv, rows, outv, sem):
        wid = lax.axis_index("s") * 2 + lax.axis_index("c")
        base = wid * CPW
        pltpu.sync_copy(idx_h.at[pl.ds(base * K, CPW * K)], idxv)

        @pl.loop(0, CPW)
        def _center(ci):
            pltpu.async_copy(feat_h.at[idxv.at[pl.ds(ci * K, K)]],
                             rows, sem).wait()
            for k in range(COUT // L):
                def mbody(j, acc):
                    return jnp.maximum(acc, rows[j, pl.ds(k * L, L)])

                acc = lax.fori_loop(1, K, mbody, rows[0, pl.ds(k * L, L)])
                outv[pl.ds(ci * COUT + k * L, L)] = acc

        pltpu.sync_copy(outv, out_h.at[pl.ds(base * COUT, CPW * COUT)])

    return gm(featT, idxf)


def kernel(xyz, points, layer_id, W0, b0, gamma0, beta0, W1, b1, gamma1,
           beta1, W2, b2, gamma2, beta2):
    del layer_id
    px, py, pz = xyz[0, 0], xyz[0, 1], xyz[0, 2]
    stride = N // S
    cx, cy, cz = px[::stride], py[::stride], pz[::stride]
    new_xyz = xyz[:, :, ::stride]

    ftT = jnp.concatenate([xyz[0], points[0]], axis=0).T
    featT = _mlp(ftT, [(W0, b0, gamma0, beta0), (W1, b1, gamma1, beta1),
                       (W2, b2, gamma2, beta2)])

    idxf = _ballquery(px, py, pz, cx, cy, cz)
    gout = _gathermax(featT, idxf)
    new_points = gout.reshape(S, COUT).T[None]
    return (new_xyz, new_points)


# speedup vs baseline: 16.9978x; 2.1632x over previous
import dataclasses
import functools

import jax
import jax.numpy as jnp
from jax import lax
from jax.experimental import pallas as pl
from jax.experimental.pallas import tpu as pltpu
from jax.experimental.pallas import tpu_sc as plsc

N = 16384
S = 2048
K = 64
R2 = 0.2 * 0.2
COUT = 256
L = 16
NW = 32
CPW = S // NW
BUF = 96


def _sc_compiler_params():
    cp = pltpu.CompilerParams()
    if "needs_layout_passes" in pltpu.CompilerParams.__dataclass_fields__:
        cp = dataclasses.replace(cp, needs_layout_passes=False)
    return cp


_RB = 2048


def _layer_body(x, w, b, g, be, out):
    c = w.shape[1]
    s = jnp.zeros((1, c), jnp.float32)
    q = jnp.zeros((1, c), jnp.float32)
    for i in range(N // _RB):
        y = jnp.dot(x[pl.ds(i * _RB, _RB), :], w[...],
                    preferred_element_type=jnp.float32,
                    precision=lax.Precision.HIGHEST) + b[...]
        out[pl.ds(i * _RB, _RB), :] = y
        s = s + jnp.sum(y, axis=0, keepdims=True)
        q = q + jnp.sum(y * y, axis=0, keepdims=True)
    mu = s * (1.0 / N)
    va = q * (1.0 / N) - mu * mu
    scale = lax.rsqrt(va + 1e-5) * g[...]
    shift = be[...] - mu * scale
    for i in range(N // _RB):
        blk = out[pl.ds(i * _RB, _RB), :]
        out[pl.ds(i * _RB, _RB), :] = jnp.maximum(blk * scale + shift, 0.0)


def _mlp(ftT, params):
    h = ftT
    for (W, b, g, be) in params:
        h = pl.pallas_call(
            _layer_body,
            out_shape=jax.ShapeDtypeStruct((N, W.shape[0]), jnp.float32),
        )(h, W.T, b.reshape(1, -1), g.reshape(1, -1), be.reshape(1, -1))
    return h


_CB = 128


def _ballprep_body(x_ref, x2_ref, w_ref, ct_ref, c2_ref, out_ref):
    mT = jnp.dot(x_ref[...], ct_ref[...],
                 preferred_element_type=jnp.float32)
    d2 = (c2_ref[...] + x2_ref[...]) - 2.0 * mT
    hit = jnp.where(d2 <= R2, w_ref[...], 0.0)
    pk = jnp.sum(hit.reshape(N // L, L, _CB), axis=1)
    out_ref[...] = pk.T.astype(jnp.int32)


def _ballprep(xpts, x2c, w2k, cT, c2r):
    return pl.pallas_call(
        _ballprep_body,
        grid=(S // _CB,),
        in_specs=[
            pl.BlockSpec((N, 3), lambda i: (0, 0)),
            pl.BlockSpec((N, 1), lambda i: (0, 0)),
            pl.BlockSpec((N, 1), lambda i: (0, 0)),
            pl.BlockSpec((3, _CB), lambda i: (0, i)),
            pl.BlockSpec((1, _CB), lambda i: (0, i)),
        ],
        out_specs=pl.BlockSpec((_CB, N // L), lambda i: (i, 0)),
        out_shape=jax.ShapeDtypeStruct((S, N // L), jnp.int32),
    )(xpts, x2c, w2k, cT, c2r)


_NC = N // L


def _ballquery(packed_flat):
    mesh = plsc.VectorSubcoreMesh(core_axis_name="c", subcore_axis_name="s")

    @functools.partial(
        pl.kernel,
        out_type=jax.ShapeDtypeStruct((S * K,), jnp.int32),
        mesh=mesh,
        scratch_types=[
            pltpu.VMEM((CPW * _NC,), jnp.int32),
            pltpu.VMEM((BUF,), jnp.int32),
            pltpu.VMEM((BUF,), jnp.int32),
            pltpu.VMEM((CPW * K,), jnp.int32),
        ],
        compiler_params=_sc_compiler_params(),
    )
    def bq(pk_h, out_h, prow, chunkbuf, buf, outv):
        wid = lax.axis_index("s") * 2 + lax.axis_index("c")
        base = wid * CPW
        pltpu.sync_copy(pk_h.at[pl.ds(base * _NC, CPW * _NC)], prow)
        lane = lax.iota(jnp.int32, 16)

        @pl.loop(0, BUF // L)
        def _zero(v):
            chunkbuf[pl.ds(v * L, L)] = jnp.zeros((L,), jnp.int32)

        @pl.loop(0, CPW)
        def _center(ci):
            rbase = ci * _NC

            def scanw(v, ccur):
                wv = prow[pl.ds(rbase + v * L, L)]
                nz = wv != 0
                plsc.store_compressed(
                    chunkbuf.at[pl.ds(jnp.minimum(ccur, K), L)],
                    lane + v * L, mask=nz)
                return ccur + plsc.all_reduce_population_count(nz)[0]

            ccur = lax.fori_loop(0, _NC // L, scanw, jnp.int32(0), unroll=2)
            nch = jnp.minimum(ccur, K)

            def unpack(r, cursor):
                cid = plsc.load_gather(chunkbuf,
                                       [jnp.full((L,), r, jnp.int32)])
                wv = plsc.load_gather(
                    prow, [jnp.full((L,), rbase, jnp.int32) + cid])
                live = jnp.full((L,), r, jnp.int32) < jnp.full(
                    (L,), nch, jnp.int32)
                hit = (((wv >> lane) & 1) != 0) & live
                plsc.store_compressed(
                    buf.at[pl.ds(jnp.minimum(cursor, K), L)],
                    cid * L + lane, mask=hit)
                return cursor + plsc.all_reduce_population_count(hit)[0]

            cursor = lax.fori_loop(0, K, unpack, jnp.int32(0))
            cnt = jnp.minimum(cursor, K)
            for k in range(K // L):
                j = lane + (k * L)
                sel = jnp.where(j < cnt, j, 0)
                outv[pl.ds(ci * K + k * L, L)] = plsc.load_gather(buf, [sel])

        pltpu.sync_copy(outv, out_h.at[pl.ds(base * K, CPW * K)])

    return bq(packed_flat)


def _gathermax(featT, idxf):
    mesh = plsc.VectorSubcoreMesh(core_axis_name="c", subcore_axis_name="s")

    @functools.partial(
        pl.kernel,
        out_type=jax.ShapeDtypeStruct((S * COUT,), jnp.float32),
        mesh=mesh,
        scratch_types=[
            pltpu.VMEM((CPW * K,), jnp.int32),
            pltpu.VMEM((K, COUT), jnp.float32),
            pltpu.VMEM((CPW * COUT,), jnp.float32),
            pltpu.SemaphoreType.DMA,
        ],
        compiler_params=_sc_compiler_params(),
    )
    def gm(feat_h, idx_h, out_h, idxv, rows, outv, sem):
        wid = lax.axis_index("s") * 2 + lax.axis_index("c")
        base = wid * CPW
        pltpu.sync_copy(idx_h.at[pl.ds(base * K, CPW * K)], idxv)

        @pl.loop(0, CPW)
        def _center(ci):
            pltpu.async_copy(feat_h.at[idxv.at[pl.ds(ci * K, K)]],
                             rows, sem).wait()
            for k in range(COUT // L):
                def mbody(j, acc):
                    return jnp.maximum(acc, rows[j, pl.ds(k * L, L)])

                acc = lax.fori_loop(1, K, mbody, rows[0, pl.ds(k * L, L)])
                outv[pl.ds(ci * COUT + k * L, L)] = acc

        pltpu.sync_copy(outv, out_h.at[pl.ds(base * COUT, CPW * COUT)])

    return gm(featT, idxf)


def kernel(xyz, points, layer_id, W0, b0, gamma0, beta0, W1, b1, gamma1,
           beta1, W2, b2, gamma2, beta2):
    del layer_id
    stride = N // S
    new_xyz = xyz[:, :, ::stride]

    ftT = jnp.concatenate([xyz[0], points[0]], axis=0).T
    featT = _mlp(ftT, [(W0, b0, gamma0, beta0), (W1, b1, gamma1, beta1),
                       (W2, b2, gamma2, beta2)])

    xpts = xyz[0].T
    x2c = jnp.sum(xpts * xpts, axis=1)[:, None]
    new_xyz_r = xpts[::stride]
    c2r = jnp.sum(new_xyz_r * new_xyz_r, axis=1)[None, :]
    w2k = jnp.left_shift(1, jnp.arange(N) % L).astype(jnp.float32)[:, None]
    packed = _ballprep(xpts, x2c, w2k, new_xyz_r.T, c2r)
    idxf = _ballquery(packed.reshape(-1))
    gout = _gathermax(featT, idxf)
    new_points = gout.reshape(S, COUT).T[None]
    return (new_xyz, new_points)


# speedup vs baseline: 29.1436x; 1.7146x over previous
import dataclasses
import functools

import jax
import jax.numpy as jnp
from jax import lax
from jax.experimental import pallas as pl
from jax.experimental.pallas import tpu as pltpu
from jax.experimental.pallas import tpu_sc as plsc

N = 16384
S = 2048
K = 64
R2 = 0.2 * 0.2
COUT = 256
L = 16
NW = 32
CPW = S // NW
BUF = 96


def _sc_compiler_params():
    cp = pltpu.CompilerParams()
    if "needs_layout_passes" in pltpu.CompilerParams.__dataclass_fields__:
        cp = dataclasses.replace(cp, needs_layout_passes=False)
    return cp


_RB = 2048


def _layer_body(x, w, b, g, be, out):
    c = w.shape[1]
    s = jnp.zeros((1, c), jnp.float32)
    q = jnp.zeros((1, c), jnp.float32)
    for i in range(N // _RB):
        y = jnp.dot(x[pl.ds(i * _RB, _RB), :], w[...],
                    preferred_element_type=jnp.float32,
                    precision=lax.Precision.HIGHEST) + b[...]
        out[pl.ds(i * _RB, _RB), :] = y
        s = s + jnp.sum(y, axis=0, keepdims=True)
        q = q + jnp.sum(y * y, axis=0, keepdims=True)
    mu = s * (1.0 / N)
    va = q * (1.0 / N) - mu * mu
    scale = lax.rsqrt(va + 1e-5) * g[...]
    shift = be[...] - mu * scale
    for i in range(N // _RB):
        blk = out[pl.ds(i * _RB, _RB), :]
        out[pl.ds(i * _RB, _RB), :] = jnp.maximum(blk * scale + shift, 0.0)


def _mlp(ftT, params):
    h = ftT
    for (W, b, g, be) in params:
        h = pl.pallas_call(
            _layer_body,
            out_shape=jax.ShapeDtypeStruct((N, W.shape[0]), jnp.float32),
        )(h, W.T, b.reshape(1, -1), g.reshape(1, -1), be.reshape(1, -1))
    return h


_CB = 128


def _ballprep_body(x_ref, x2_ref, w_ref, ct_ref, c2_ref, out_ref):
    mT = jnp.dot(x_ref[...], ct_ref[...],
                 preferred_element_type=jnp.float32)
    d2 = (c2_ref[...] + x2_ref[...]) - 2.0 * mT
    hit = jnp.where(d2 <= R2, w_ref[...], 0.0)
    pk = jnp.sum(hit.reshape(N // L, L, _CB), axis=1)
    out_ref[...] = pk.T.astype(jnp.int32)


def _ballprep(xpts, x2c, w2k, cT, c2r):
    return pl.pallas_call(
        _ballprep_body,
        grid=(S // _CB,),
        in_specs=[
            pl.BlockSpec((N, 3), lambda i: (0, 0)),
            pl.BlockSpec((N, 1), lambda i: (0, 0)),
            pl.BlockSpec((N, 1), lambda i: (0, 0)),
            pl.BlockSpec((3, _CB), lambda i: (0, i)),
            pl.BlockSpec((1, _CB), lambda i: (0, i)),
        ],
        out_specs=pl.BlockSpec((_CB, N // L), lambda i: (i, 0)),
        out_shape=jax.ShapeDtypeStruct((S, N // L), jnp.int32),
    )(xpts, x2c, w2k, cT, c2r)


_NC = N // L


def _ballquery(packed_flat):
    mesh = plsc.VectorSubcoreMesh(core_axis_name="c", subcore_axis_name="s")

    @functools.partial(
        pl.kernel,
        out_type=jax.ShapeDtypeStruct((S * K,), jnp.int32),
        mesh=mesh,
        scratch_types=[
            pltpu.VMEM((CPW * _NC,), jnp.int32),
            pltpu.VMEM((BUF,), jnp.int32),
            pltpu.VMEM((BUF,), jnp.int32),
            pltpu.VMEM((CPW * K,), jnp.int32),
        ],
        compiler_params=_sc_compiler_params(),
    )
    def bq(pk_h, out_h, prow, chunkbuf, buf, outv):
        wid = lax.axis_index("s") * 2 + lax.axis_index("c")
        base = wid * CPW
        pltpu.sync_copy(pk_h.at[pl.ds(base * _NC, CPW * _NC)], prow)
        lane = lax.iota(jnp.int32, 16)

        @pl.loop(0, BUF // L)
        def _zero(v):
            chunkbuf[pl.ds(v * L, L)] = jnp.zeros((L,), jnp.int32)

        @pl.loop(0, CPW)
        def _center(ci):
            rbase = ci * _NC

            def scanw(v, ccur):
                wv = prow[pl.ds(rbase + v * L, L)]
                nz = wv != 0
                plsc.store_compressed(
                    chunkbuf.at[pl.ds(jnp.minimum(ccur, K), L)],
                    lane + v * L, mask=nz)
                return ccur + plsc.all_reduce_population_count(nz)[0]

            ccur = lax.fori_loop(0, _NC // L, scanw, jnp.int32(0), unroll=2)
            nch = jnp.minimum(ccur, K)

            def unpack(r, cursor):
                cid = plsc.load_gather(chunkbuf,
                                       [jnp.full((L,), r, jnp.int32)])
                wv = plsc.load_gather(
                    prow, [jnp.full((L,), rbase, jnp.int32) + cid])
                live = jnp.full((L,), r, jnp.int32) < jnp.full(
                    (L,), nch, jnp.int32)
                hit = (((wv >> lane) & 1) != 0) & live
                plsc.store_compressed(
                    buf.at[pl.ds(jnp.minimum(cursor, K), L)],
                    cid * L + lane, mask=hit)
                return cursor + plsc.all_reduce_population_count(hit)[0]

            cursor = lax.fori_loop(0, K, unpack, jnp.int32(0))
            cnt = jnp.minimum(cursor, K)
            for k in range(K // L):
                j = lane + (k * L)
                sel = jnp.where(j < cnt, j, 0)
                outv[pl.ds(ci * K + k * L, L)] = plsc.load_gather(buf, [sel])

        pltpu.sync_copy(outv, out_h.at[pl.ds(base * K, CPW * K)])

    return bq(packed_flat)


_G = 2
_NG = CPW // _G


def _gathermax(featT, idxf):
    mesh = plsc.VectorSubcoreMesh(core_axis_name="c", subcore_axis_name="s")

    @functools.partial(
        pl.kernel,
        out_type=jax.ShapeDtypeStruct((S * COUT,), jnp.float32),
        mesh=mesh,
        scratch_types=[
            pltpu.VMEM((CPW * K,), jnp.int32),
            pltpu.VMEM((_G * K, COUT), jnp.float32),
            pltpu.VMEM((_G * K, COUT), jnp.float32),
            pltpu.VMEM((CPW * COUT,), jnp.float32),
            pltpu.SemaphoreType.DMA,
            pltpu.SemaphoreType.DMA,
        ],
        compiler_params=_sc_compiler_params(),
    )
    def gm(feat_h, idx_h, out_h, idxv, rowsA, rowsB, outv, semA, semB):
        wid = lax.axis_index("s") * 2 + lax.axis_index("c")
        base = wid * CPW
        pltpu.sync_copy(idx_h.at[pl.ds(base * K, CPW * K)], idxv)

        def fire(g, rbuf, sem):
            pltpu.async_copy(
                feat_h.at[idxv.at[pl.ds(g * (_G * K), _G * K)]], rbuf, sem)

        def drain(rbuf, sem):
            pltpu.make_async_copy(feat_h.at[pl.ds(0, _G * K), :], rbuf,
                                  sem).wait()

        def process(g, rbuf):
            for lc in range(_G):
                def mbody(j, acc):
                    return tuple(
                        jnp.maximum(acc[k],
                                    rbuf[lc * K + j, pl.ds(k * L, L)])
                        for k in range(COUT // L))

                acc0 = tuple(rbuf[lc * K, pl.ds(k * L, L)]
                             for k in range(COUT // L))
                acc = lax.fori_loop(1, K, mbody, acc0)
                ob = (g * _G + lc) * COUT
                for k in range(COUT // L):
                    outv[pl.ds(ob + k * L, L)] = acc[k]

        fire(0, rowsA, semA)

        @pl.loop(0, _NG, step=2)
        def _grp(g):
            fire(jnp.minimum(g + 1, _NG - 1), rowsB, semB)
            drain(rowsA, semA)
            process(g, rowsA)
            fire(jnp.minimum(g + 2, _NG - 1), rowsA, semA)
            drain(rowsB, semB)
            process(g + 1, rowsB)

        drain(rowsA, semA)
        pltpu.sync_copy(outv, out_h.at[pl.ds(base * COUT, CPW * COUT)])

    return gm(featT, idxf)


def kernel(xyz, points, layer_id, W0, b0, gamma0, beta0, W1, b1, gamma1,
           beta1, W2, b2, gamma2, beta2):
    del layer_id
    stride = N // S
    new_xyz = xyz[:, :, ::stride]

    ftT = jnp.concatenate([xyz[0], points[0]], axis=0).T
    featT = _mlp(ftT, [(W0, b0, gamma0, beta0), (W1, b1, gamma1, beta1),
                       (W2, b2, gamma2, beta2)])

    xpts = xyz[0].T
    x2c = jnp.sum(xpts * xpts, axis=1)[:, None]
    new_xyz_r = xpts[::stride]
    c2r = jnp.sum(new_xyz_r * new_xyz_r, axis=1)[None, :]
    w2k = jnp.left_shift(1, jnp.arange(N) % L).astype(jnp.float32)[:, None]
    packed = _ballprep(xpts, x2c, w2k, new_xyz_r.T, c2r)
    idxf = _ballquery(packed.reshape(-1))
    gout = _gathermax(featT, idxf)
    new_points = gout.reshape(S, COUT).T[None]
    return (new_xyz, new_points)
